# Initial kernel scaffold; baseline (speedup 1.0000x reference)
import jax
import jax.numpy as jnp
from jax.experimental import pallas as pl


def kernel(word_ids, emb_weight):
    raise NotImplementedError("write your pallas kernel here")


# speedup vs baseline: 5.7615x; 5.7615x over previous
import functools

import jax
import jax.numpy as jnp
from jax import lax
from jax.experimental import pallas as pl
from jax.experimental.pallas import tpu as pltpu
from jax.experimental.pallas import tpu_sc as plsc


def _make_gather(V, D, N, NC, NS, CH):
    NW = NC * NS
    n_per_w = N // NW
    n_ch = n_per_w // CH
    mesh = plsc.VectorSubcoreMesh(core_axis_name="c", subcore_axis_name="s")

    @functools.partial(
        pl.kernel,
        out_type=jax.ShapeDtypeStruct((N, D), jnp.float32),
        mesh=mesh,
        scratch_types=[
            pltpu.VMEM((n_ch, CH), jnp.int32),
            pltpu.VMEM((CH, D), jnp.float32),
            pltpu.SemaphoreType.DMA,
        ],
    )
    def gather_kernel(idx_hbm, table_hbm, out_hbm, idx_v, rows_v, sem):
        wid = lax.axis_index("s") * NC + lax.axis_index("c")
        base = wid * n_per_w
        pltpu.sync_copy(idx_hbm.at[wid], idx_v)

        @pl.loop(0, n_ch)
        def _(c):
            pltpu.async_copy(table_hbm.at[idx_v.at[c]], rows_v, sem).wait()
            pltpu.sync_copy(rows_v, out_hbm.at[pl.ds(base + c * CH, CH)])

    return gather_kernel


def kernel(word_ids, emb_weight):
    B, L = word_ids.shape
    V, D = emb_weight.shape
    N = B * L
    info = plsc.get_sparse_core_info()
    NC, NS = info.num_cores, info.num_subcores
    NW = NC * NS
    CH = 128
    idx = word_ids.reshape(-1).astype(jnp.int32).reshape(NW, N // NW // CH, CH)
    out = _make_gather(V, D, N, NC, NS, CH)(idx, emb_weight)
    return out.reshape(B, L, D)


# speedup vs baseline: 8.0150x; 1.3911x over previous
import functools

import jax
import jax.numpy as jnp
from jax import lax
from jax.experimental import pallas as pl
from jax.experimental.pallas import tpu as pltpu
from jax.experimental.pallas import tpu_sc as plsc


def _make_gather(V, D, N, NC, NS, CH, NBUF):
    NW = NC * NS
    n_per_w = N // NW
    n_ch = n_per_w // CH
    assert n_ch % NBUF == 0 and n_ch >= 2 * NBUF
    mesh = plsc.VectorSubcoreMesh(core_axis_name="c", subcore_axis_name="s")

    @functools.partial(
        pl.kernel,
        out_type=jax.ShapeDtypeStruct((N, D), jnp.float32),
        mesh=mesh,
        scratch_types=[
            pltpu.VMEM((n_ch, CH), jnp.int32),
            pltpu.VMEM((NBUF, CH, D), jnp.float32),
            [pltpu.SemaphoreType.DMA] * NBUF,
            [pltpu.SemaphoreType.DMA] * NBUF,
        ],
    )
    def gather_kernel(idx_hbm, table_hbm, out_hbm, idx_v, rows_v, gsems, osems):
        wid = lax.axis_index("s") * NC + lax.axis_index("c")
        base = wid * n_per_w
        pltpu.sync_copy(idx_hbm.at[wid], idx_v)

        def gather_start(c, b):
            pltpu.async_copy(table_hbm.at[idx_v.at[c]], rows_v.at[b], gsems[b])

        def gather_wait(c, b):
            pltpu.make_async_copy(
                table_hbm.at[idx_v.at[c]], rows_v.at[b], gsems[b]
            ).wait()

        def out_start(c, b):
            pltpu.async_copy(
                rows_v.at[b], out_hbm.at[pl.ds(base + c * CH, CH)], osems[b]
            )

        def out_wait(c, b):
            pltpu.make_async_copy(
                rows_v.at[b], out_hbm.at[pl.ds(base + c * CH, CH)], osems[b]
            ).wait()

        for b in range(NBUF):
            gather_start(b, b)

        @pl.loop(0, n_ch - NBUF, step=NBUF)
        def _(c0):
            for b in range(NBUF):
                c = c0 + b
                gather_wait(c, b)
                out_start(c, b)
                out_wait(c, b)
                gather_start(c + NBUF, b)

        for b in range(NBUF):
            c = n_ch - NBUF + b
            gather_wait(c, b)
            out_start(c, b)
            out_wait(c, b)

    return gather_kernel


def kernel(word_ids, emb_weight):
    B, L = word_ids.shape
    V, D = emb_weight.shape
    N = B * L
    info = plsc.get_sparse_core_info()
    NC, NS = info.num_cores, info.num_subcores
    NW = NC * NS
    CH = 128
    NBUF = 5
    idx = word_ids.reshape(-1).astype(jnp.int32).reshape(NW, N // NW // CH, CH)
    out = _make_gather(V, D, N, NC, NS, CH, NBUF)(idx, emb_weight)
    return out.reshape(B, L, D)
